# baseline (speedup 1.0000x reference)
import jax
import jax.numpy as jnp
from jax.experimental import pallas as pl
from jax.experimental.pallas import tpu as pltpu


def _mm_kernel(a_ref, h_ref, o_ref, acc_ref):
    @pl.when(pl.program_id(1) == 0)
    def _init():
        acc_ref[...] = jnp.zeros_like(acc_ref)

    acc_ref[...] += jnp.dot(a_ref[...], h_ref[...],
                            preferred_element_type=jnp.float32)

    @pl.when(pl.program_id(1) == pl.num_programs(1) - 1)
    def _emit():
        o_ref[...] = acc_ref[...].astype(o_ref.dtype)


def _mm_epilogue_kernel(a_ref, h_ref, bd0_ref, bd1_ref, o_ref, acc_ref):
    @pl.when(pl.program_id(1) == 0)
    def _init():
        acc_ref[...] = jnp.zeros_like(acc_ref)

    acc_ref[...] += jnp.dot(a_ref[...], h_ref[...],
                            preferred_element_type=jnp.float32)

    @pl.when(pl.program_id(1) == pl.num_programs(1) - 1)
    def _apply_weights():
        t = jnp.dot(acc_ref[...].astype(jnp.bfloat16), bd0_ref[...],
                    preferred_element_type=jnp.float32)
        o_ref[...] = jnp.dot(t.astype(jnp.bfloat16), bd1_ref[...],
                             preferred_element_type=jnp.float32)


def kernel(x, adj, W0, W1):
    B, N, D = x.shape
    C = B * D
    TI = 512
    TK = 512

    adj_bf = adj.astype(jnp.bfloat16)
    xt = jnp.transpose(x, (1, 0, 2)).reshape(N, C).astype(jnp.bfloat16)
    eye = jnp.eye(B, dtype=jnp.bfloat16)
    bd0 = jnp.kron(eye, W0.T.astype(jnp.bfloat16))
    bd1 = jnp.kron(eye, W1.T.astype(jnp.bfloat16))

    grid = (N // TI, N // TK)
    a_spec = pl.BlockSpec((TI, TK), lambda i, k: (i, k))
    h_spec = pl.BlockSpec((TK, C), lambda i, k: (k, 0))
    o_spec = pl.BlockSpec((TI, C), lambda i, k: (i, 0))
    w_spec = pl.BlockSpec((C, C), lambda i, k: (0, 0))
    acc = pltpu.VMEM((TI, C), jnp.float32)

    g = pl.pallas_call(
        _mm_kernel,
        grid=grid,
        in_specs=[a_spec, h_spec],
        out_specs=o_spec,
        out_shape=jax.ShapeDtypeStruct((N, C), jnp.bfloat16),
        scratch_shapes=[acc],
    )(adj_bf, xt)

    out_flat = pl.pallas_call(
        _mm_epilogue_kernel,
        grid=grid,
        in_specs=[a_spec, h_spec, w_spec, w_spec],
        out_specs=o_spec,
        out_shape=jax.ShapeDtypeStruct((N, C), jnp.float32),
        scratch_shapes=[acc],
    )(adj_bf, g, bd0, bd1)

    return jnp.transpose(out_flat.reshape(N, B, D), (1, 0, 2))


# speedup vs baseline: 2.3840x; 2.3840x over previous
import jax
import jax.numpy as jnp
from jax.experimental import pallas as pl


def _layer_kernel(a_ref, h_ref, o_ref):
    a_bf = a_ref[...].astype(jnp.bfloat16)
    o_ref[...] = jnp.dot(a_bf, h_ref[...],
                         preferred_element_type=jnp.float32).astype(o_ref.dtype)


def _layer_epilogue_kernel(a_ref, h_ref, bd0_ref, bd1_ref, o_ref):
    a_bf = a_ref[...].astype(jnp.bfloat16)
    acc = jnp.dot(a_bf, h_ref[...], preferred_element_type=jnp.float32)
    t = jnp.dot(acc.astype(jnp.bfloat16), bd0_ref[...],
                preferred_element_type=jnp.float32)
    o_ref[...] = jnp.dot(t.astype(jnp.bfloat16), bd1_ref[...],
                         preferred_element_type=jnp.float32)


def kernel(x, adj, W0, W1):
    B, N, D = x.shape
    C = B * D
    TI = 512

    xt = jnp.transpose(x, (1, 0, 2)).reshape(N, C).astype(jnp.bfloat16)
    eye = jnp.eye(B, dtype=jnp.bfloat16)
    bd0 = jnp.kron(eye, W0.T.astype(jnp.bfloat16))
    bd1 = jnp.kron(eye, W1.T.astype(jnp.bfloat16))

    grid = (N // TI,)
    a_spec = pl.BlockSpec((TI, N), lambda i: (i, 0))
    h_spec = pl.BlockSpec((N, C), lambda i: (0, 0))
    o_spec = pl.BlockSpec((TI, C), lambda i: (i, 0))
    w_spec = pl.BlockSpec((C, C), lambda i: (0, 0))

    g = pl.pallas_call(
        _layer_kernel,
        grid=grid,
        in_specs=[a_spec, h_spec],
        out_specs=o_spec,
        out_shape=jax.ShapeDtypeStruct((N, C), jnp.bfloat16),
    )(adj, xt)

    out_flat = pl.pallas_call(
        _layer_epilogue_kernel,
        grid=grid,
        in_specs=[a_spec, h_spec, w_spec, w_spec],
        out_specs=o_spec,
        out_shape=jax.ShapeDtypeStruct((N, C), jnp.float32),
    )(adj, g, bd0, bd1)

    return jnp.transpose(out_flat.reshape(N, B, D), (1, 0, 2))
